# baseline (speedup 1.0000x reference)
import functools

import jax
import jax.numpy as jnp
from jax import lax
from jax.experimental import pallas as pl
from jax.experimental.pallas import tpu as pltpu
from jax.experimental.pallas import tpu_sc as plsc

N1, N2, N3 = 642, 2562, 10242
HID = 192
IMG = 224.0
CAM_F, CAM_C = 248.0, 112.0

_BN = 256


def _pad_to(x, m, axis):
    n = x.shape[axis]
    r = (-n) % m
    if r == 0:
        return x
    pads = [(0, 0)] * x.ndim
    pads[axis] = (0, r)
    return jnp.pad(x, pads)


def _mm2_body(x_ref, w0_ref, w1_ref, o0_ref, o1_ref):
    x = x_ref[...]
    o0_ref[...] = jnp.dot(x, w0_ref[...], preferred_element_type=jnp.float32)
    o1_ref[...] = jnp.dot(x, w1_ref[...], preferred_element_type=jnp.float32)


@functools.partial(jax.jit, static_argnames=())
def _mm2(x, w0, w1):
    n, f = x.shape
    h = w0.shape[1]
    xp = _pad_to(_pad_to(x, _BN, 0), 128, 1)
    w0p = _pad_to(w0, 128, 0)
    w1p = _pad_to(w1, 128, 0)
    npad, fp = xp.shape
    grid = (npad // _BN,)
    out = pl.pallas_call(
        _mm2_body,
        grid=grid,
        in_specs=[
            pl.BlockSpec((_BN, fp), lambda i: (i, 0)),
            pl.BlockSpec((fp, h), lambda i: (0, 0)),
            pl.BlockSpec((fp, h), lambda i: (0, 0)),
        ],
        out_specs=[
            pl.BlockSpec((_BN, h), lambda i: (i, 0)),
            pl.BlockSpec((_BN, h), lambda i: (i, 0)),
        ],
        out_shape=[
            jax.ShapeDtypeStruct((npad, h), jnp.float32),
            jax.ShapeDtypeStruct((npad, h), jnp.float32),
        ],
    )(xp, w0p, w1p)
    return out[0][:n], out[1][:n]


def _mm1_body(x_ref, w_ref, o_ref):
    o_ref[...] = jnp.dot(x_ref[...], w_ref[...], preferred_element_type=jnp.float32)


def _mm1(x, w):
    n, f = x.shape
    h = w.shape[1]
    xp = _pad_to(_pad_to(x, _BN, 0), 128, 1)
    wp = _pad_to(_pad_to(w, 128, 0), 128, 1)
    npad, fp = xp.shape
    hp = wp.shape[1]
    out = pl.pallas_call(
        _mm1_body,
        grid=(npad // _BN,),
        in_specs=[
            pl.BlockSpec((_BN, fp), lambda i: (i, 0)),
            pl.BlockSpec((fp, hp), lambda i: (0, 0)),
        ],
        out_specs=pl.BlockSpec((_BN, hp), lambda i: (i, 0)),
        out_shape=jax.ShapeDtypeStruct((npad, hp), jnp.float32),
    )(xp, wp)
    return out[:n, :h]


_CH = 1024


@functools.lru_cache(maxsize=None)
def _sc_segsum(NP, EP, RPP):
    mesh = plsc.VectorSubcoreMesh(core_axis_name="c", subcore_axis_name="s")
    n_pass = 6 // RPP

    @functools.partial(
        pl.kernel, mesh=mesh,
        out_type=jax.ShapeDtypeStruct((HID * NP,), jnp.float32),
        compiler_params=pltpu.CompilerParams(needs_layout_passes=False),
        scratch_types=[
            pltpu.VMEM((2, _CH,), jnp.int32),
            pltpu.VMEM((2, _CH,), jnp.int32),
            pltpu.VMEM((RPP * NP,), jnp.float32),
            pltpu.VMEM((RPP * NP,), jnp.float32),
            pltpu.SemaphoreType.DMA,
            pltpu.SemaphoreType.DMA,
        ],
    )
    def k(yT, edges, out, sbuf, dbuf, yrow, orow, sem0, sem1):
        wid = lax.axis_index("s") * 2 + lax.axis_index("c")
        sems = (sem0, sem1)
        n_chunk = EP // _CH
        for p in range(n_pass):
            base = (wid * 6 + p * RPP) * NP
            pltpu.sync_copy(yT.at[pl.ds(base, RPP * NP)], yrow)

            @plsc.parallel_loop(0, RPP * NP // 16, unroll=4)
            def zbody(i):
                orow[pl.ds(i * 16, 16)] = jnp.zeros((16,), jnp.float32)

            for b in range(2):
                pltpu.async_copy(edges.at[pl.ds(b * _CH, _CH)], sbuf.at[b], sems[b])
                pltpu.async_copy(edges.at[pl.ds(EP + b * _CH, _CH)], dbuf.at[b], sems[b])

            def pairbody(q, _):
                for b in range(2):
                    c = q * 2 + b
                    pltpu.make_async_copy(edges.at[pl.ds(0, _CH)], sbuf.at[b], sems[b]).wait()
                    pltpu.make_async_copy(edges.at[pl.ds(0, _CH)], dbuf.at[b], sems[b]).wait()

                    yr = [yrow.at[pl.ds(r * NP, NP)] for r in range(RPP)]
                    orr = [orow.at[pl.ds(r * NP, NP)] for r in range(RPP)]

                    @plsc.parallel_loop(0, _CH // 64, unroll=2)
                    def jbody(j):
                        for u in range(4):
                            o = j * 64 + u * 16
                            s_v = sbuf[b, pl.ds(o, 16)]
                            d_v = dbuf[b, pl.ds(o, 16)]
                            for r in range(RPP):
                                v = plsc.load_gather(yr[r], [s_v])
                                plsc.addupdate_scatter(orr[r], [d_v], v)

                    @pl.when(c + 2 < n_chunk)
                    def _():
                        nc = (c + 2) * _CH
                        pltpu.async_copy(edges.at[pl.ds(nc, _CH)], sbuf.at[b], sems[b])
                        pltpu.async_copy(edges.at[pl.ds(EP + nc, _CH)], dbuf.at[b], sems[b])
                return 0
            lax.fori_loop(0, n_chunk // 2, pairbody, 0)
            pltpu.sync_copy(orow, out.at[pl.ds(base, RPP * NP)])

    return k


@functools.lru_cache(maxsize=None)
def _sc_degree(NP, EP):
    mesh = plsc.VectorSubcoreMesh(core_axis_name="c", subcore_axis_name="s")

    @functools.partial(
        pl.kernel, mesh=mesh,
        out_type=jax.ShapeDtypeStruct((NP,), jnp.float32),
        compiler_params=pltpu.CompilerParams(needs_layout_passes=False),
        scratch_types=[
            pltpu.VMEM((_CH,), jnp.int32),
            pltpu.VMEM((NP,), jnp.float32),
        ],
    )
    def k(edges, out, dbuf, acc):
        wid = lax.axis_index("s") * 2 + lax.axis_index("c")

        @pl.when(wid == 0)
        def _():
            def zbody(i, _):
                acc[pl.ds(i * 16, 16)] = jnp.zeros((16,), jnp.float32)
                return 0
            lax.fori_loop(0, NP // 16, zbody, 0)

            ones = jnp.ones((16,), jnp.float32)

            def cbody(c, _):
                pltpu.sync_copy(edges.at[pl.ds(EP + c * _CH, _CH)], dbuf)

                @plsc.parallel_loop(0, _CH // 16, unroll=4)
                def jbody(j):
                    d_v = dbuf[pl.ds(j * 16, 16)]
                    plsc.addupdate_scatter(acc, [d_v], ones)
                return 0
            lax.fori_loop(0, EP // _CH, cbody, 0)
            pltpu.sync_copy(acc, out)

    return k


_SC_HW = (112 * 112, 56 * 56, 28 * 28, 14 * 14)
_SC_CH = (64, 128, 256, 512)
_SC_ROW0 = (0, 64, 192, 448)
_SC_CPT = (2, 4, 8, 16)
_SC_PSTR = tuple((3 * hw + 7) // 8 * 8 for hw in _SC_HW)


@functools.lru_cache(maxsize=None)
def _sc_bilinear(NPp):
    mesh = plsc.VectorSubcoreMesh(core_axis_name="c", subcore_axis_name="s")

    @functools.partial(
        pl.kernel, mesh=mesh,
        out_type=jax.ShapeDtypeStruct((960 * NPp,), jnp.float32),
        compiler_params=pltpu.CompilerParams(needs_layout_passes=False),
        scratch_types=[
            pltpu.VMEM((3 * _SC_HW[0],), jnp.float32),
            pltpu.VMEM((4 * NPp,), jnp.int32),
            pltpu.VMEM((4 * NPp,), jnp.float32),
            pltpu.VMEM((NPp,), jnp.float32),
        ],
    )
    def k(p0, p1, p2, p3, i0, i1, i2, i3, w0, w1, w2, w3, out,
          pbuf, ibuf, wbuf, obuf):
        wid = lax.axis_index("s") * 2 + lax.axis_index("c")
        planes = (p0, p1, p2, p3)
        idxs = (i0, i1, i2, i3)
        ws = (w0, w1, w2, w3)
        for s in range(4):
            HW3 = _SC_PSTR[s]
            cs = _SC_CPT[s]
            pltpu.sync_copy(idxs[s], ibuf)
            pltpu.sync_copy(ws[s], wbuf)
            for j in range(cs):
                ch = wid * cs + j
                row = _SC_ROW0[s] + ch
                pltpu.sync_copy(planes[s].at[pl.ds(ch * HW3, HW3)],
                                pbuf.at[pl.ds(0, HW3)])

                @plsc.parallel_loop(0, NPp // 16, unroll=4)
                def ibody(i):
                    o = i * 16
                    acc = jnp.zeros((16,), jnp.float32)
                    for kk in range(4):
                        iv = ibuf[pl.ds(kk * NPp + o, 16)]
                        wv = wbuf[pl.ds(kk * NPp + o, 16)]
                        acc = acc + wv * plsc.load_gather(pbuf, [iv])
                    obuf[pl.ds(o, 16)] = acc
                pltpu.sync_copy(obuf, out.at[pl.ds(row * NPp, NPp)])

    return k


def _proj_tables(pts, assign, NPp):
    n = pts.shape[0]
    Z = jnp.clip(pts[:, 2] + 1.0, 0.2, None)
    u = CAM_F * pts[:, 0] / Z + CAM_C
    v = CAM_F * pts[:, 1] / Z + CAM_C
    base = assign.astype(jnp.int32)
    idx_all, w_all = [], []
    for s in range(4):
        H = W = (112, 56, 28, 14)[s]
        sc = H / IMG
        xs = jnp.clip(u * sc, 0.0, W - 1.0)
        ys = jnp.clip(v * sc, 0.0, H - 1.0)
        x0 = jnp.floor(xs)
        y0 = jnp.floor(ys)
        wx1 = xs - x0
        wx0 = 1.0 - wx1
        wy1 = ys - y0
        wy0 = 1.0 - wy1
        xi0 = x0.astype(jnp.int32)
        yi0 = y0.astype(jnp.int32)
        xi1 = jnp.minimum(xi0 + 1, W - 1)
        yi1 = jnp.minimum(yi0 + 1, H - 1)
        vb = base * (H * W)
        ia = vb + yi0 * W + xi0
        ib = vb + yi1 * W + xi0
        ic = vb + yi0 * W + xi1
        id_ = vb + yi1 * W + xi1
        wa = wx0 * wy0
        wb = wx0 * wy1
        wc = wx1 * wy0
        wd = wx1 * wy1
        pad = NPp - n
        idx = jnp.concatenate([jnp.pad(a, (0, pad)) for a in (ia, ib, ic, id_)])
        w = jnp.concatenate([jnp.pad(a, (0, pad)) for a in (wa, wb, wc, wd)])
        idx_all.append(idx)
        w_all.append(w)
    return idx_all, w_all


def _assigned_proj_sc(pts, planes, assign):
    n = pts.shape[0]
    NPp = _round_up(n, 16)
    idx_all, w_all = _proj_tables(pts, assign, NPp)
    out = _sc_bilinear(NPp)(planes[0], planes[1], planes[2], planes[3],
                            idx_all[0], idx_all[1], idx_all[2], idx_all[3],
                            w_all[0], w_all[1], w_all[2], w_all[3])
    feat = out.reshape(960, NPp)[:, :n].T
    return jnp.concatenate([feat, pts], axis=1)


def _round_up(v, m):
    return v + (-v) % m


def _pack_edges(src, dst, n, EP):
    E = src.shape[0]
    pad = jnp.full((EP - E,), n, jnp.int32)
    return jnp.concatenate([src, pad, dst, pad])


def _seg_mean_sc(y1, packed_edges, deg_inv, n, NP, EP, RPP):
    h = y1.shape[1]
    yT = jnp.pad(y1.T, ((0, HID - h), (0, NP - n)))
    out_flat = _sc_segsum(NP, EP, RPP)(yT.reshape(-1), packed_edges)
    outT = out_flat.reshape(HID, NP)
    return outT[:h, :n].T * deg_inv[:, None]


def _gconv(x, W0, W1, b, lvl, relu=False):
    packed, deg_inv, n, NP, EP, RPP = lvl
    y1 = _mm1(x, W1)
    agg = _seg_mean_sc(y1, packed, deg_inv, n, NP, EP, RPP)
    y0 = _mm1(x, W0)
    out = y0 + b + agg
    return jax.nn.relu(out) if relu else out


def _gbottleneck(x, p, lvl):
    Win0, Win1, bin_, blkW, blkb, Wout0, Wout1, bout = p
    h = _gconv(x, Win0, Win1, bin_, lvl, relu=True)
    for i in range(6):
        t = _gconv(h, blkW[i, 0, 0], blkW[i, 0, 1], blkb[i, 0], lvl, relu=True)
        t = _gconv(t, blkW[i, 1, 0], blkW[i, 1, 1], blkb[i, 1], lvl, relu=True)
        h = (h + t) * 0.5
    out = _gconv(h, Wout0, Wout1, bout, lvl)
    return out, h


def _bilinear(fm, x, y):
    C, H, W = fm.shape
    x = jnp.clip(x, 0.0, W - 1.0)
    y = jnp.clip(y, 0.0, H - 1.0)
    x0 = jnp.floor(x)
    y0 = jnp.floor(y)
    wx1 = x - x0
    wx0 = 1.0 - wx1
    wy1 = y - y0
    wy0 = 1.0 - wy1
    xi0 = x0.astype(jnp.int32)
    yi0 = y0.astype(jnp.int32)
    xi1 = jnp.minimum(xi0 + 1, W - 1)
    yi1 = jnp.minimum(yi0 + 1, H - 1)
    va = fm[:, yi0, xi0]
    vb = fm[:, yi1, xi0]
    vc = fm[:, yi0, xi1]
    vd = fm[:, yi1, xi1]
    out = va * (wx0 * wy0) + vb * (wx0 * wy1) + vc * (wx1 * wy0) + vd * (wx1 * wy1)
    return out.T


def _project_points(pts, fmaps):
    Z = jnp.clip(pts[:, 2] + 1.0, 0.2, None)
    u = CAM_F * pts[:, 0] / Z + CAM_C
    v = CAM_F * pts[:, 1] / Z + CAM_C
    feats = []
    for fm in fmaps:
        s = fm.shape[1] / IMG
        feats.append(_bilinear(fm, u * s, v * s))
    feats.append(pts)
    return jnp.concatenate(feats, axis=1)


def _assigned_proj(pts, fmaps_views, assign, num_views=3):
    out = 0.0
    for vi in range(num_views):
        fmaps = [fs[vi] for fs in fmaps_views]
        feat = _project_points(pts, fmaps)
        mask = (assign == vi).astype(feat.dtype)[:, None]
        out = out + feat * mask
    return out


def _encoder(imgs, enc_params):
    feats = []
    x = imgs
    for (W, b) in enc_params:
        co, ci = W.shape[0], W.shape[1]
        patches = lax.conv_general_dilated_patches(
            x, (3, 3), (2, 2), 'SAME',
            dimension_numbers=('NCHW', 'OIHW', 'NCHW'))
        nb, f, ho, wo = patches.shape
        pm = patches.reshape(nb, f, ho * wo).transpose(0, 2, 1).reshape(nb * ho * wo, f)
        y = _mm1(pm, W.reshape(co, f).T)
        y = jax.nn.relu(y + b)
        x = y.reshape(nb, ho, wo, co).transpose(0, 3, 1, 2)
        feats.append(x)
    return feats


def _unpool(x, up):
    mid = (x[up[:, 0]] + x[up[:, 1]]) * 0.5
    return jnp.concatenate([x, mid], axis=0)


def _make_level(adj, n, RPP):
    src, dst = adj[0], adj[1]
    NP = _round_up(n + 1, 16)
    EP = _round_up(src.shape[0], 2 * _CH)
    packed = _pack_edges(src, dst, n, EP)
    deg = _sc_degree(NP, EP)(packed)[:n]
    deg_inv = 1.0 / jnp.maximum(deg, 1.0)
    return (packed, deg_inv, n, NP, EP, RPP)


def kernel(img, proj, depth_values, init_pts, enc_params, gcn0, gcn1, gcn2,
           fin, pa0, pa1, adj1, adj2, adj3, up1, up2):
    imgs = img[0]
    fmaps = _encoder(imgs, enc_params)
    planes = [
        _pad_to(fm.transpose(1, 0, 2, 3).reshape(fm.shape[1], -1), _SC_PSTR[s], 1)[:, :_SC_PSTR[s]].reshape(-1)
        for s, fm in enumerate(fmaps)
    ]
    a0 = pa0[0]
    a1 = pa1[0]

    lvl1 = _make_level(adj1, N1, 6)
    lvl2 = _make_level(adj2, N2, 6)
    lvl3 = _make_level(adj3, N3, 6)

    x = _assigned_proj_sc(init_pts, planes, a0)
    x1, xh = _gbottleneck(x, gcn0, lvl1)
    x1 = x1 + init_pts
    x1_up = _unpool(x1, up1)

    x = _assigned_proj_sc(x1, planes, a0)
    x = _unpool(jnp.concatenate([x, xh], axis=1), up1)
    x2, xh = _gbottleneck(x, gcn1, lvl2)
    x2 = x2 + x1_up
    x2_up = _unpool(x2, up2)

    x = _assigned_proj_sc(x2, planes, a1)
    x = _unpool(jnp.concatenate([x, xh], axis=1), up2)
    x3, _ = _gbottleneck(x, gcn2, lvl3)
    x3 = jax.nn.relu(x3)
    x3 = _gconv(x3, fin[0], fin[1], fin[2], lvl3)
    x3 = x3 + x2_up
    return (x1, x2, x3, x1_up, x2_up)


# speedup vs baseline: 1.0177x; 1.0177x over previous
import functools

import jax
import jax.numpy as jnp
from jax import lax
from jax.experimental import pallas as pl
from jax.experimental.pallas import tpu as pltpu
from jax.experimental.pallas import tpu_sc as plsc

N1, N2, N3 = 642, 2562, 10242
HID = 192
IMG = 224.0
CAM_F, CAM_C = 248.0, 112.0

_BN = 256


def _pad_to(x, m, axis):
    n = x.shape[axis]
    r = (-n) % m
    if r == 0:
        return x
    pads = [(0, 0)] * x.ndim
    pads[axis] = (0, r)
    return jnp.pad(x, pads)


def _mm2_body(x_ref, w0_ref, w1_ref, o0_ref, o1_ref):
    x = x_ref[...]
    o0_ref[...] = jnp.dot(x, w0_ref[...], preferred_element_type=jnp.float32)
    o1_ref[...] = jnp.dot(x, w1_ref[...], preferred_element_type=jnp.float32)


@functools.partial(jax.jit, static_argnames=())
def _mm2(x, w0, w1):
    n, f = x.shape
    h = w0.shape[1]
    xp = _pad_to(_pad_to(x, _BN, 0), 128, 1)
    w0p = _pad_to(w0, 128, 0)
    w1p = _pad_to(w1, 128, 0)
    npad, fp = xp.shape
    grid = (npad // _BN,)
    out = pl.pallas_call(
        _mm2_body,
        grid=grid,
        in_specs=[
            pl.BlockSpec((_BN, fp), lambda i: (i, 0)),
            pl.BlockSpec((fp, h), lambda i: (0, 0)),
            pl.BlockSpec((fp, h), lambda i: (0, 0)),
        ],
        out_specs=[
            pl.BlockSpec((_BN, h), lambda i: (i, 0)),
            pl.BlockSpec((_BN, h), lambda i: (i, 0)),
        ],
        out_shape=[
            jax.ShapeDtypeStruct((npad, h), jnp.float32),
            jax.ShapeDtypeStruct((npad, h), jnp.float32),
        ],
    )(xp, w0p, w1p)
    return out[0][:n], out[1][:n]


def _mm1_body(x_ref, w_ref, o_ref):
    o_ref[...] = jnp.dot(x_ref[...], w_ref[...], preferred_element_type=jnp.float32)


def _mm1(x, w):
    n, f = x.shape
    h = w.shape[1]
    xp = _pad_to(_pad_to(x, _BN, 0), 128, 1)
    wp = _pad_to(_pad_to(w, 128, 0), 128, 1)
    npad, fp = xp.shape
    hp = wp.shape[1]
    out = pl.pallas_call(
        _mm1_body,
        grid=(npad // _BN,),
        in_specs=[
            pl.BlockSpec((_BN, fp), lambda i: (i, 0)),
            pl.BlockSpec((fp, hp), lambda i: (0, 0)),
        ],
        out_specs=pl.BlockSpec((_BN, hp), lambda i: (i, 0)),
        out_shape=jax.ShapeDtypeStruct((npad, hp), jnp.float32),
    )(xp, wp)
    return out[:n, :h]


_CH = 1024


@functools.lru_cache(maxsize=None)
def _sc_segsum(NP, EP, RPP):
    mesh = plsc.VectorSubcoreMesh(core_axis_name="c", subcore_axis_name="s")
    n_pass = 6 // RPP

    @functools.partial(
        pl.kernel, mesh=mesh,
        out_type=jax.ShapeDtypeStruct((HID * NP,), jnp.float32),
        compiler_params=pltpu.CompilerParams(needs_layout_passes=False),
        scratch_types=[
            pltpu.VMEM((2, _CH,), jnp.int32),
            pltpu.VMEM((2, _CH,), jnp.int32),
            pltpu.VMEM((RPP * NP,), jnp.float32),
            pltpu.VMEM((RPP * NP,), jnp.float32),
            pltpu.SemaphoreType.DMA,
            pltpu.SemaphoreType.DMA,
        ],
    )
    def k(yT, edges, out, sbuf, dbuf, yrow, orow, sem0, sem1):
        wid = lax.axis_index("s") * 2 + lax.axis_index("c")
        sems = (sem0, sem1)
        n_chunk = EP // _CH
        for p in range(n_pass):
            base = (wid * 6 + p * RPP) * NP
            pltpu.sync_copy(yT.at[pl.ds(base, RPP * NP)], yrow)

            @plsc.parallel_loop(0, RPP * NP // 16, unroll=4)
            def zbody(i):
                orow[pl.ds(i * 16, 16)] = jnp.zeros((16,), jnp.float32)

            for b in range(2):
                pltpu.async_copy(edges.at[pl.ds(b * _CH, _CH)], sbuf.at[b], sems[b])
                pltpu.async_copy(edges.at[pl.ds(EP + b * _CH, _CH)], dbuf.at[b], sems[b])

            def pairbody(q, _):
                for b in range(2):
                    c = q * 2 + b
                    pltpu.make_async_copy(edges.at[pl.ds(0, _CH)], sbuf.at[b], sems[b]).wait()
                    pltpu.make_async_copy(edges.at[pl.ds(0, _CH)], dbuf.at[b], sems[b]).wait()

                    yr = [yrow.at[pl.ds(r * NP, NP)] for r in range(RPP)]
                    orr = [orow.at[pl.ds(r * NP, NP)] for r in range(RPP)]

                    @plsc.parallel_loop(0, _CH // 64, unroll=2)
                    def jbody(j):
                        for u in range(4):
                            o = j * 64 + u * 16
                            s_v = sbuf[b, pl.ds(o, 16)]
                            d_v = dbuf[b, pl.ds(o, 16)]
                            for r in range(RPP):
                                v = plsc.load_gather(yr[r], [s_v])
                                plsc.addupdate_scatter(orr[r], [d_v], v)

                    @pl.when(c + 2 < n_chunk)
                    def _():
                        nc = (c + 2) * _CH
                        pltpu.async_copy(edges.at[pl.ds(nc, _CH)], sbuf.at[b], sems[b])
                        pltpu.async_copy(edges.at[pl.ds(EP + nc, _CH)], dbuf.at[b], sems[b])
                return 0
            lax.fori_loop(0, n_chunk // 2, pairbody, 0)
            pltpu.sync_copy(orow, out.at[pl.ds(base, RPP * NP)])

    return k


@functools.lru_cache(maxsize=None)
def _sc_degree(NP, EP):
    mesh = plsc.VectorSubcoreMesh(core_axis_name="c", subcore_axis_name="s")

    @functools.partial(
        pl.kernel, mesh=mesh,
        out_type=jax.ShapeDtypeStruct((NP,), jnp.float32),
        compiler_params=pltpu.CompilerParams(needs_layout_passes=False),
        scratch_types=[
            pltpu.VMEM((_CH,), jnp.int32),
            pltpu.VMEM((NP,), jnp.float32),
        ],
    )
    def k(edges, out, dbuf, acc):
        wid = lax.axis_index("s") * 2 + lax.axis_index("c")

        @pl.when(wid == 0)
        def _():
            def zbody(i, _):
                acc[pl.ds(i * 16, 16)] = jnp.zeros((16,), jnp.float32)
                return 0
            lax.fori_loop(0, NP // 16, zbody, 0)

            ones = jnp.ones((16,), jnp.float32)

            def cbody(c, _):
                pltpu.sync_copy(edges.at[pl.ds(EP + c * _CH, _CH)], dbuf)

                @plsc.parallel_loop(0, _CH // 16, unroll=4)
                def jbody(j):
                    d_v = dbuf[pl.ds(j * 16, 16)]
                    plsc.addupdate_scatter(acc, [d_v], ones)
                return 0
            lax.fori_loop(0, EP // _CH, cbody, 0)
            pltpu.sync_copy(acc, out)

    return k


_SC_HW = (112 * 112, 56 * 56, 28 * 28, 14 * 14)
_SC_CH = (64, 128, 256, 512)
_SC_ROW0 = (0, 64, 192, 448)
_SC_CPT = (2, 4, 8, 16)
_SC_PSTR = tuple((3 * hw + 7) // 8 * 8 for hw in _SC_HW)


@functools.lru_cache(maxsize=None)
def _sc_bilinear(NPp):
    mesh = plsc.VectorSubcoreMesh(core_axis_name="c", subcore_axis_name="s")

    @functools.partial(
        pl.kernel, mesh=mesh,
        out_type=jax.ShapeDtypeStruct((960 * NPp,), jnp.float32),
        compiler_params=pltpu.CompilerParams(needs_layout_passes=False),
        scratch_types=[
            pltpu.VMEM((3 * _SC_HW[0],), jnp.float32),
            pltpu.VMEM((4 * NPp,), jnp.int32),
            pltpu.VMEM((4 * NPp,), jnp.float32),
            pltpu.VMEM((NPp,), jnp.float32),
        ],
    )
    def k(p0, p1, p2, p3, i0, i1, i2, i3, w0, w1, w2, w3, out,
          pbuf, ibuf, wbuf, obuf):
        wid = lax.axis_index("s") * 2 + lax.axis_index("c")
        planes = (p0, p1, p2, p3)
        idxs = (i0, i1, i2, i3)
        ws = (w0, w1, w2, w3)
        for s in range(4):
            HW3 = _SC_PSTR[s]
            cs = _SC_CPT[s]
            pltpu.sync_copy(idxs[s], ibuf)
            pltpu.sync_copy(ws[s], wbuf)
            for j in range(cs):
                ch = wid * cs + j
                row = _SC_ROW0[s] + ch
                pltpu.sync_copy(planes[s].at[pl.ds(ch * HW3, HW3)],
                                pbuf.at[pl.ds(0, HW3)])

                @plsc.parallel_loop(0, NPp // 16, unroll=4)
                def ibody(i):
                    o = i * 16
                    acc = jnp.zeros((16,), jnp.float32)
                    for kk in range(4):
                        iv = ibuf[pl.ds(kk * NPp + o, 16)]
                        wv = wbuf[pl.ds(kk * NPp + o, 16)]
                        acc = acc + wv * plsc.load_gather(pbuf, [iv])
                    obuf[pl.ds(o, 16)] = acc
                pltpu.sync_copy(obuf, out.at[pl.ds(row * NPp, NPp)])

    return k


def _proj_tables(pts, assign, NPp):
    n = pts.shape[0]
    Z = jnp.clip(pts[:, 2] + 1.0, 0.2, None)
    u = CAM_F * pts[:, 0] / Z + CAM_C
    v = CAM_F * pts[:, 1] / Z + CAM_C
    base = assign.astype(jnp.int32)
    idx_all, w_all = [], []
    for s in range(4):
        H = W = (112, 56, 28, 14)[s]
        sc = H / IMG
        xs = jnp.clip(u * sc, 0.0, W - 1.0)
        ys = jnp.clip(v * sc, 0.0, H - 1.0)
        x0 = jnp.floor(xs)
        y0 = jnp.floor(ys)
        wx1 = xs - x0
        wx0 = 1.0 - wx1
        wy1 = ys - y0
        wy0 = 1.0 - wy1
        xi0 = x0.astype(jnp.int32)
        yi0 = y0.astype(jnp.int32)
        xi1 = jnp.minimum(xi0 + 1, W - 1)
        yi1 = jnp.minimum(yi0 + 1, H - 1)
        vb = base * (H * W)
        ia = vb + yi0 * W + xi0
        ib = vb + yi1 * W + xi0
        ic = vb + yi0 * W + xi1
        id_ = vb + yi1 * W + xi1
        wa = wx0 * wy0
        wb = wx0 * wy1
        wc = wx1 * wy0
        wd = wx1 * wy1
        pad = NPp - n
        idx = jnp.concatenate([jnp.pad(a, (0, pad)) for a in (ia, ib, ic, id_)])
        w = jnp.concatenate([jnp.pad(a, (0, pad)) for a in (wa, wb, wc, wd)])
        idx_all.append(idx)
        w_all.append(w)
    return idx_all, w_all


def _assigned_proj_sc(pts, planes, assign):
    n = pts.shape[0]
    NPp = _round_up(n, 16)
    idx_all, w_all = _proj_tables(pts, assign, NPp)
    out = _sc_bilinear(NPp)(planes[0], planes[1], planes[2], planes[3],
                            idx_all[0], idx_all[1], idx_all[2], idx_all[3],
                            w_all[0], w_all[1], w_all[2], w_all[3])
    feat = out.reshape(960, NPp)[:, :n].T
    return jnp.concatenate([feat, pts], axis=1)


def _round_up(v, m):
    return v + (-v) % m


def _pack_edges(src, dst, n, EP):
    E = src.shape[0]
    pad = jnp.full((EP - E,), n, jnp.int32)
    return jnp.concatenate([src, pad, dst, pad])


def _seg_mean_sc(y1, packed_edges, deg_inv, n, NP, EP, RPP):
    h = y1.shape[1]
    yT = jnp.pad(y1.T, ((0, HID - h), (0, NP - n)))
    out_flat = _sc_segsum(NP, EP, RPP)(yT.reshape(-1), packed_edges)
    outT = out_flat.reshape(HID, NP)
    return outT[:h, :n].T * deg_inv[:, None]


def _gconv(x, W0, W1, b, lvl, relu=False):
    packed, deg_inv, n, NP, EP, RPP = lvl
    y1 = _mm1(x, W1)
    agg = _seg_mean_sc(y1, packed, deg_inv, n, NP, EP, RPP)
    y0 = _mm1(x, W0)
    out = y0 + b + agg
    return jax.nn.relu(out) if relu else out


def _gbottleneck(x, p, lvl):
    Win0, Win1, bin_, blkW, blkb, Wout0, Wout1, bout = p
    h = _gconv(x, Win0, Win1, bin_, lvl, relu=True)
    for i in range(6):
        t = _gconv(h, blkW[i, 0, 0], blkW[i, 0, 1], blkb[i, 0], lvl, relu=True)
        t = _gconv(t, blkW[i, 1, 0], blkW[i, 1, 1], blkb[i, 1], lvl, relu=True)
        h = (h + t) * 0.5
    out = _gconv(h, Wout0, Wout1, bout, lvl)
    return out, h


def _bilinear(fm, x, y):
    C, H, W = fm.shape
    x = jnp.clip(x, 0.0, W - 1.0)
    y = jnp.clip(y, 0.0, H - 1.0)
    x0 = jnp.floor(x)
    y0 = jnp.floor(y)
    wx1 = x - x0
    wx0 = 1.0 - wx1
    wy1 = y - y0
    wy0 = 1.0 - wy1
    xi0 = x0.astype(jnp.int32)
    yi0 = y0.astype(jnp.int32)
    xi1 = jnp.minimum(xi0 + 1, W - 1)
    yi1 = jnp.minimum(yi0 + 1, H - 1)
    va = fm[:, yi0, xi0]
    vb = fm[:, yi1, xi0]
    vc = fm[:, yi0, xi1]
    vd = fm[:, yi1, xi1]
    out = va * (wx0 * wy0) + vb * (wx0 * wy1) + vc * (wx1 * wy0) + vd * (wx1 * wy1)
    return out.T


def _project_points(pts, fmaps):
    Z = jnp.clip(pts[:, 2] + 1.0, 0.2, None)
    u = CAM_F * pts[:, 0] / Z + CAM_C
    v = CAM_F * pts[:, 1] / Z + CAM_C
    feats = []
    for fm in fmaps:
        s = fm.shape[1] / IMG
        feats.append(_bilinear(fm, u * s, v * s))
    feats.append(pts)
    return jnp.concatenate(feats, axis=1)


def _assigned_proj(pts, fmaps_views, assign, num_views=3):
    out = 0.0
    for vi in range(num_views):
        fmaps = [fs[vi] for fs in fmaps_views]
        feat = _project_points(pts, fmaps)
        mask = (assign == vi).astype(feat.dtype)[:, None]
        out = out + feat * mask
    return out


def _encoder(imgs, enc_params):
    feats = []
    x = imgs
    for (W, b) in enc_params:
        co, ci = W.shape[0], W.shape[1]
        patches = lax.conv_general_dilated_patches(
            x, (3, 3), (2, 2), 'SAME',
            dimension_numbers=('NCHW', 'OIHW', 'NCHW'))
        nb, f, ho, wo = patches.shape
        pm = patches.reshape(nb, f, ho * wo).reshape(nb * f, ho * wo).T.reshape(nb, ho * wo, f) if False else patches.transpose(0, 2, 3, 1).reshape(nb * ho * wo, f)
        y = _mm1(pm, W.reshape(co, f).T)
        y = jax.nn.relu(y + b)
        x = y.reshape(nb, ho, wo, co).transpose(0, 3, 1, 2)
        feats.append(x)
    return feats


def _unpool(x, up):
    mid = (x[up[:, 0]] + x[up[:, 1]]) * 0.5
    return jnp.concatenate([x, mid], axis=0)


def _make_level(adj, n, RPP):
    src, dst = adj[0], adj[1]
    NP = _round_up(n + 1, 16)
    EP = _round_up(src.shape[0], 2 * _CH)
    packed = _pack_edges(src, dst, n, EP)
    deg = _sc_degree(NP, EP)(packed)[:n]
    deg_inv = 1.0 / jnp.maximum(deg, 1.0)
    return (packed, deg_inv, n, NP, EP, RPP)


def kernel(img, proj, depth_values, init_pts, enc_params, gcn0, gcn1, gcn2,
           fin, pa0, pa1, adj1, adj2, adj3, up1, up2):
    imgs = img[0]
    fmaps = _encoder(imgs, enc_params)
    planes = [
        _pad_to(fm.transpose(1, 0, 2, 3).reshape(fm.shape[1], -1), _SC_PSTR[s], 1)[:, :_SC_PSTR[s]].reshape(-1)
        for s, fm in enumerate(fmaps)
    ]
    a0 = pa0[0]
    a1 = pa1[0]

    lvl1 = _make_level(adj1, N1, 6)
    lvl2 = _make_level(adj2, N2, 6)
    lvl3 = _make_level(adj3, N3, 6)

    x = _assigned_proj_sc(init_pts, planes, a0)
    x1, xh = _gbottleneck(x, gcn0, lvl1)
    x1 = x1 + init_pts
    x1_up = _unpool(x1, up1)

    x = _assigned_proj_sc(x1, planes, a0)
    x = _unpool(jnp.concatenate([x, xh], axis=1), up1)
    x2, xh = _gbottleneck(x, gcn1, lvl2)
    x2 = x2 + x1_up
    x2_up = _unpool(x2, up2)

    x = _assigned_proj_sc(x2, planes, a1)
    x = _unpool(jnp.concatenate([x, xh], axis=1), up2)
    x3, _ = _gbottleneck(x, gcn2, lvl3)
    x3 = jax.nn.relu(x3)
    x3 = _gconv(x3, fin[0], fin[1], fin[2], lvl3)
    x3 = x3 + x2_up
    return (x1, x2, x3, x1_up, x2_up)


# speedup vs baseline: 1.1033x; 1.0841x over previous
import functools

import jax
import jax.numpy as jnp
from jax import lax
from jax.experimental import pallas as pl
from jax.experimental.pallas import tpu as pltpu
from jax.experimental.pallas import tpu_sc as plsc

N1, N2, N3 = 642, 2562, 10242
HID = 192
IMG = 224.0
CAM_F, CAM_C = 248.0, 112.0

_BN = 256


def _pad_to(x, m, axis):
    n = x.shape[axis]
    r = (-n) % m
    if r == 0:
        return x
    pads = [(0, 0)] * x.ndim
    pads[axis] = (0, r)
    return jnp.pad(x, pads)


def _mm2_body(x_ref, w0_ref, w1_ref, o0_ref, o1_ref):
    x = x_ref[...]
    o0_ref[...] = jnp.dot(x, w0_ref[...], preferred_element_type=jnp.float32)
    o1_ref[...] = jnp.dot(x, w1_ref[...], preferred_element_type=jnp.float32)


@functools.partial(jax.jit, static_argnames=())
def _mm2(x, w0, w1):
    n, f = x.shape
    h = w0.shape[1]
    xp = _pad_to(_pad_to(x, _BN, 0), 128, 1)
    w0p = _pad_to(w0, 128, 0)
    w1p = _pad_to(w1, 128, 0)
    npad, fp = xp.shape
    grid = (npad // _BN,)
    out = pl.pallas_call(
        _mm2_body,
        grid=grid,
        in_specs=[
            pl.BlockSpec((_BN, fp), lambda i: (i, 0)),
            pl.BlockSpec((fp, h), lambda i: (0, 0)),
            pl.BlockSpec((fp, h), lambda i: (0, 0)),
        ],
        out_specs=[
            pl.BlockSpec((_BN, h), lambda i: (i, 0)),
            pl.BlockSpec((_BN, h), lambda i: (i, 0)),
        ],
        out_shape=[
            jax.ShapeDtypeStruct((npad, h), jnp.float32),
            jax.ShapeDtypeStruct((npad, h), jnp.float32),
        ],
    )(xp, w0p, w1p)
    return out[0][:n], out[1][:n]


def _mm1_body(x_ref, w_ref, o_ref):
    o_ref[...] = jnp.dot(x_ref[...], w_ref[...], preferred_element_type=jnp.float32)


def _mm1(x, w):
    n, f = x.shape
    h = w.shape[1]
    xp = _pad_to(_pad_to(x, _BN, 0), 128, 1)
    wp = _pad_to(_pad_to(w, 128, 0), 128, 1)
    npad, fp = xp.shape
    hp = wp.shape[1]
    out = pl.pallas_call(
        _mm1_body,
        grid=(npad // _BN,),
        in_specs=[
            pl.BlockSpec((_BN, fp), lambda i: (i, 0)),
            pl.BlockSpec((fp, hp), lambda i: (0, 0)),
        ],
        out_specs=pl.BlockSpec((_BN, hp), lambda i: (i, 0)),
        out_shape=jax.ShapeDtypeStruct((npad, hp), jnp.float32),
    )(xp, wp)
    return out[:n, :h]


_CH = 1024


@functools.lru_cache(maxsize=None)
def _sc_segsum(NP, EP, RPP):
    mesh = plsc.VectorSubcoreMesh(core_axis_name="c", subcore_axis_name="s")
    n_pass = 6 // RPP

    @functools.partial(
        pl.kernel, mesh=mesh,
        out_type=jax.ShapeDtypeStruct((HID * NP,), jnp.float32),
        compiler_params=pltpu.CompilerParams(needs_layout_passes=False),
        scratch_types=[
            pltpu.VMEM((2, _CH,), jnp.int32),
            pltpu.VMEM((2, _CH,), jnp.int32),
            pltpu.VMEM((RPP * NP,), jnp.float32),
            pltpu.VMEM((RPP * NP,), jnp.float32),
            pltpu.SemaphoreType.DMA,
            pltpu.SemaphoreType.DMA,
        ],
    )
    def k(yT, edges, out, sbuf, dbuf, yrow, orow, sem0, sem1):
        wid = lax.axis_index("s") * 2 + lax.axis_index("c")
        sems = (sem0, sem1)
        n_chunk = EP // _CH
        for p in range(n_pass):
            base = (wid * 6 + p * RPP) * NP
            pltpu.sync_copy(yT.at[pl.ds(base, RPP * NP)], yrow)

            @plsc.parallel_loop(0, RPP * NP // 16, unroll=4)
            def zbody(i):
                orow[pl.ds(i * 16, 16)] = jnp.zeros((16,), jnp.float32)

            for b in range(2):
                pltpu.async_copy(edges.at[pl.ds(b * _CH, _CH)], sbuf.at[b], sems[b])
                pltpu.async_copy(edges.at[pl.ds(EP + b * _CH, _CH)], dbuf.at[b], sems[b])

            def pairbody(q, _):
                for b in range(2):
                    c = q * 2 + b
                    pltpu.make_async_copy(edges.at[pl.ds(0, _CH)], sbuf.at[b], sems[b]).wait()
                    pltpu.make_async_copy(edges.at[pl.ds(0, _CH)], dbuf.at[b], sems[b]).wait()

                    yr = [yrow.at[pl.ds(r * NP, NP)] for r in range(RPP)]
                    orr = [orow.at[pl.ds(r * NP, NP)] for r in range(RPP)]

                    @plsc.parallel_loop(0, _CH // 64, unroll=2)
                    def jbody(j):
                        for u in range(4):
                            o = j * 64 + u * 16
                            s_v = sbuf[b, pl.ds(o, 16)]
                            d_v = dbuf[b, pl.ds(o, 16)]
                            for r in range(RPP):
                                v = plsc.load_gather(yr[r], [s_v])
                                plsc.addupdate_scatter(orr[r], [d_v], v)

                    @pl.when(c + 2 < n_chunk)
                    def _():
                        nc = (c + 2) * _CH
                        pltpu.async_copy(edges.at[pl.ds(nc, _CH)], sbuf.at[b], sems[b])
                        pltpu.async_copy(edges.at[pl.ds(EP + nc, _CH)], dbuf.at[b], sems[b])
                return 0
            lax.fori_loop(0, n_chunk // 2, pairbody, 0)
            pltpu.sync_copy(orow, out.at[pl.ds(base, RPP * NP)])

    return k


@functools.lru_cache(maxsize=None)
def _sc_degree(NP, EP):
    mesh = plsc.VectorSubcoreMesh(core_axis_name="c", subcore_axis_name="s")

    @functools.partial(
        pl.kernel, mesh=mesh,
        out_type=jax.ShapeDtypeStruct((NP,), jnp.float32),
        compiler_params=pltpu.CompilerParams(needs_layout_passes=False),
        scratch_types=[
            pltpu.VMEM((_CH,), jnp.int32),
            pltpu.VMEM((NP,), jnp.float32),
        ],
    )
    def k(edges, out, dbuf, acc):
        wid = lax.axis_index("s") * 2 + lax.axis_index("c")

        @pl.when(wid == 0)
        def _():
            def zbody(i, _):
                acc[pl.ds(i * 16, 16)] = jnp.zeros((16,), jnp.float32)
                return 0
            lax.fori_loop(0, NP // 16, zbody, 0)

            ones = jnp.ones((16,), jnp.float32)

            def cbody(c, _):
                pltpu.sync_copy(edges.at[pl.ds(EP + c * _CH, _CH)], dbuf)

                @plsc.parallel_loop(0, _CH // 16, unroll=4)
                def jbody(j):
                    d_v = dbuf[pl.ds(j * 16, 16)]
                    plsc.addupdate_scatter(acc, [d_v], ones)
                return 0
            lax.fori_loop(0, EP // _CH, cbody, 0)
            pltpu.sync_copy(acc, out)

    return k


_SC_HW = (112 * 112, 56 * 56, 28 * 28, 14 * 14)
_SC_CH = (64, 128, 256, 512)
_SC_ROW0 = (0, 64, 192, 448)
_SC_CPT = (2, 4, 8, 16)
_SC_PSTR = tuple((3 * hw + 7) // 8 * 8 for hw in _SC_HW)


@functools.lru_cache(maxsize=None)
def _sc_bilinear(NPp):
    mesh = plsc.VectorSubcoreMesh(core_axis_name="c", subcore_axis_name="s")

    @functools.partial(
        pl.kernel, mesh=mesh,
        out_type=jax.ShapeDtypeStruct((960 * NPp,), jnp.float32),
        compiler_params=pltpu.CompilerParams(needs_layout_passes=False),
        scratch_types=[
            pltpu.VMEM((3 * _SC_HW[0],), jnp.float32),
            pltpu.VMEM((4 * NPp,), jnp.int32),
            pltpu.VMEM((4 * NPp,), jnp.float32),
            pltpu.VMEM((NPp,), jnp.float32),
        ],
    )
    def k(p0, p1, p2, p3, i0, i1, i2, i3, w0, w1, w2, w3, out,
          pbuf, ibuf, wbuf, obuf):
        wid = lax.axis_index("s") * 2 + lax.axis_index("c")
        planes = (p0, p1, p2, p3)
        idxs = (i0, i1, i2, i3)
        ws = (w0, w1, w2, w3)
        for s in range(4):
            HW3 = _SC_PSTR[s]
            cs = _SC_CPT[s]
            pltpu.sync_copy(idxs[s], ibuf)
            pltpu.sync_copy(ws[s], wbuf)
            for j in range(cs):
                ch = wid * cs + j
                row = _SC_ROW0[s] + ch
                pltpu.sync_copy(planes[s].at[pl.ds(ch * HW3, HW3)],
                                pbuf.at[pl.ds(0, HW3)])

                @plsc.parallel_loop(0, NPp // 16, unroll=4)
                def ibody(i):
                    o = i * 16
                    acc = jnp.zeros((16,), jnp.float32)
                    for kk in range(4):
                        iv = ibuf[pl.ds(kk * NPp + o, 16)]
                        wv = wbuf[pl.ds(kk * NPp + o, 16)]
                        acc = acc + wv * plsc.load_gather(pbuf, [iv])
                    obuf[pl.ds(o, 16)] = acc
                pltpu.sync_copy(obuf, out.at[pl.ds(row * NPp, NPp)])

    return k


def _proj_tables(pts, assign, NPp):
    n = pts.shape[0]
    Z = jnp.clip(pts[:, 2] + 1.0, 0.2, None)
    u = CAM_F * pts[:, 0] / Z + CAM_C
    v = CAM_F * pts[:, 1] / Z + CAM_C
    base = assign.astype(jnp.int32)
    idx_all, w_all = [], []
    for s in range(4):
        H = W = (112, 56, 28, 14)[s]
        sc = H / IMG
        xs = jnp.clip(u * sc, 0.0, W - 1.0)
        ys = jnp.clip(v * sc, 0.0, H - 1.0)
        x0 = jnp.floor(xs)
        y0 = jnp.floor(ys)
        wx1 = xs - x0
        wx0 = 1.0 - wx1
        wy1 = ys - y0
        wy0 = 1.0 - wy1
        xi0 = x0.astype(jnp.int32)
        yi0 = y0.astype(jnp.int32)
        xi1 = jnp.minimum(xi0 + 1, W - 1)
        yi1 = jnp.minimum(yi0 + 1, H - 1)
        vb = base * (H * W)
        ia = vb + yi0 * W + xi0
        ib = vb + yi1 * W + xi0
        ic = vb + yi0 * W + xi1
        id_ = vb + yi1 * W + xi1
        wa = wx0 * wy0
        wb = wx0 * wy1
        wc = wx1 * wy0
        wd = wx1 * wy1
        pad = NPp - n
        idx = jnp.concatenate([jnp.pad(a, (0, pad)) for a in (ia, ib, ic, id_)])
        w = jnp.concatenate([jnp.pad(a, (0, pad)) for a in (wa, wb, wc, wd)])
        idx_all.append(idx)
        w_all.append(w)
    return idx_all, w_all


def _assigned_proj_sc(pts, planes, assign):
    n = pts.shape[0]
    NPp = _round_up(n, 16)
    idx_all, w_all = _proj_tables(pts, assign, NPp)
    out = _sc_bilinear(NPp)(planes[0], planes[1], planes[2], planes[3],
                            idx_all[0], idx_all[1], idx_all[2], idx_all[3],
                            w_all[0], w_all[1], w_all[2], w_all[3])
    feat = out.reshape(960, NPp)[:, :n].T
    return jnp.concatenate([feat, pts], axis=1)


def _round_up(v, m):
    return v + (-v) % m


def _pack_edges(src, dst, n, EP):
    E = src.shape[0]
    pad = jnp.full((EP - E,), n, jnp.int32)
    return jnp.concatenate([src, pad, dst, pad])


def _seg_mean_sc(y1, packed_edges, deg_inv, n, NP, EP, RPP):
    h = y1.shape[1]
    yT = jnp.pad(y1.T, ((0, HID - h), (0, NP - n)))
    out_flat = _sc_segsum(NP, EP, RPP)(yT.reshape(-1), packed_edges)
    outT = out_flat.reshape(HID, NP)
    return outT[:h, :n].T * deg_inv[:, None]


def _gconv(x, W0, W1, b, lvl, relu=False):
    packed, deg_inv, n, NP, EP, RPP = lvl
    y1 = _mm1(x, W1)
    agg = _seg_mean_sc(y1, packed, deg_inv, n, NP, EP, RPP)
    y0 = _mm1(x, W0)
    out = y0 + b + agg
    return jax.nn.relu(out) if relu else out


def _gbottleneck(x, p, lvl):
    Win0, Win1, bin_, blkW, blkb, Wout0, Wout1, bout = p
    h = _gconv(x, Win0, Win1, bin_, lvl, relu=True)
    for i in range(6):
        t = _gconv(h, blkW[i, 0, 0], blkW[i, 0, 1], blkb[i, 0], lvl, relu=True)
        t = _gconv(t, blkW[i, 1, 0], blkW[i, 1, 1], blkb[i, 1], lvl, relu=True)
        h = (h + t) * 0.5
    out = _gconv(h, Wout0, Wout1, bout, lvl)
    return out, h


def _bilinear(fm, x, y):
    C, H, W = fm.shape
    x = jnp.clip(x, 0.0, W - 1.0)
    y = jnp.clip(y, 0.0, H - 1.0)
    x0 = jnp.floor(x)
    y0 = jnp.floor(y)
    wx1 = x - x0
    wx0 = 1.0 - wx1
    wy1 = y - y0
    wy0 = 1.0 - wy1
    xi0 = x0.astype(jnp.int32)
    yi0 = y0.astype(jnp.int32)
    xi1 = jnp.minimum(xi0 + 1, W - 1)
    yi1 = jnp.minimum(yi0 + 1, H - 1)
    va = fm[:, yi0, xi0]
    vb = fm[:, yi1, xi0]
    vc = fm[:, yi0, xi1]
    vd = fm[:, yi1, xi1]
    out = va * (wx0 * wy0) + vb * (wx0 * wy1) + vc * (wx1 * wy0) + vd * (wx1 * wy1)
    return out.T


def _project_points(pts, fmaps):
    Z = jnp.clip(pts[:, 2] + 1.0, 0.2, None)
    u = CAM_F * pts[:, 0] / Z + CAM_C
    v = CAM_F * pts[:, 1] / Z + CAM_C
    feats = []
    for fm in fmaps:
        s = fm.shape[1] / IMG
        feats.append(_bilinear(fm, u * s, v * s))
    feats.append(pts)
    return jnp.concatenate(feats, axis=1)


def _assigned_proj(pts, fmaps_views, assign, num_views=3):
    out = 0.0
    for vi in range(num_views):
        fmaps = [fs[vi] for fs in fmaps_views]
        feat = _project_points(pts, fmaps)
        mask = (assign == vi).astype(feat.dtype)[:, None]
        out = out + feat * mask
    return out


def _encoder(imgs, enc_params):
    feats = []
    x = imgs
    for (W, b) in enc_params:
        x = jax.nn.relu(lax.conv_general_dilated(
            x, W, (2, 2), 'SAME',
            dimension_numbers=('NCHW', 'OIHW', 'NCHW')) + b[None, :, None, None])
        feats.append(x)
    return feats


def _unpool(x, up):
    mid = (x[up[:, 0]] + x[up[:, 1]]) * 0.5
    return jnp.concatenate([x, mid], axis=0)


def _make_level(adj, n, RPP):
    src, dst = adj[0], adj[1]
    NP = _round_up(n + 1, 16)
    EP = _round_up(src.shape[0], 2 * _CH)
    packed = _pack_edges(src, dst, n, EP)
    deg = _sc_degree(NP, EP)(packed)[:n]
    deg_inv = 1.0 / jnp.maximum(deg, 1.0)
    return (packed, deg_inv, n, NP, EP, RPP)


def kernel(img, proj, depth_values, init_pts, enc_params, gcn0, gcn1, gcn2,
           fin, pa0, pa1, adj1, adj2, adj3, up1, up2):
    imgs = img[0]
    fmaps = _encoder(imgs, enc_params)
    planes = [
        _pad_to(fm.transpose(1, 0, 2, 3).reshape(fm.shape[1], -1), _SC_PSTR[s], 1)[:, :_SC_PSTR[s]].reshape(-1)
        for s, fm in enumerate(fmaps)
    ]
    a0 = pa0[0]
    a1 = pa1[0]

    lvl1 = _make_level(adj1, N1, 6)
    lvl2 = _make_level(adj2, N2, 6)
    lvl3 = _make_level(adj3, N3, 6)

    x = _assigned_proj_sc(init_pts, planes, a0)
    x1, xh = _gbottleneck(x, gcn0, lvl1)
    x1 = x1 + init_pts
    x1_up = _unpool(x1, up1)

    x = _assigned_proj_sc(x1, planes, a0)
    x = _unpool(jnp.concatenate([x, xh], axis=1), up1)
    x2, xh = _gbottleneck(x, gcn1, lvl2)
    x2 = x2 + x1_up
    x2_up = _unpool(x2, up2)

    x = _assigned_proj_sc(x2, planes, a1)
    x = _unpool(jnp.concatenate([x, xh], axis=1), up2)
    x3, _ = _gbottleneck(x, gcn2, lvl3)
    x3 = jax.nn.relu(x3)
    x3 = _gconv(x3, fin[0], fin[1], fin[2], lvl3)
    x3 = x3 + x2_up
    return (x1, x2, x3, x1_up, x2_up)
